# baseline (speedup 1.0000x reference)
import functools

import jax
import jax.numpy as jnp
from jax import lax
from jax.experimental import pallas as pl
from jax.experimental.pallas import tpu as pltpu
from jax.experimental.pallas import tpu_sc as plsc

C, T, H, W = 3, 64, 384, 384
TS = T // 4
FRAME = H * W
QUARTER = FRAME // 4
NW = 32
PIECES = C * TS * 4
PER_W = PIECES // NW


def _sc_gather(frames_flat):
    mesh = plsc.VectorSubcoreMesh(core_axis_name="c", subcore_axis_name="s")

    @functools.partial(
        pl.kernel,
        mesh=mesh,
        out_type=jax.ShapeDtypeStruct((C * TS * FRAME,), jnp.float32),
        scratch_types=[pltpu.SemaphoreType.DMA],
    )
    def k(src, out, sem):
        wid = lax.axis_index("s") * 2 + lax.axis_index("c")
        copies = []
        for j in range(PER_W):
            p = wid * PER_W + j
            slab = p // 4
            q = p % 4
            c = slab // TS
            i = slab % TS
            src_off = (c * T + (i * 21) // 5) * FRAME + q * QUARTER
            copies.append(
                pltpu.make_async_copy(
                    src.at[pl.ds(src_off, QUARTER)],
                    out.at[pl.ds(p * QUARTER, QUARTER)],
                    sem,
                )
            )
        for cp in copies:
            cp.start()
        for cp in copies:
            cp.wait()

    return k(frames_flat)


def kernel(frames):
    slow = _sc_gather(frames.reshape(-1)).reshape(C, TS, H, W)
    return (slow, frames)


# speedup vs baseline: 4.3206x; 4.3206x over previous
import functools

import jax
import jax.numpy as jnp
from jax import lax
from jax.experimental import pallas as pl
from jax.experimental.pallas import tpu as pltpu
from jax.experimental.pallas import tpu_sc as plsc

C, T, H, W = 3, 64, 384, 384
TS = T // 4
FRAME = H * W
QUARTER = FRAME // 4
NW = 32
PIECES = C * TS * 4
PER_W = PIECES // NW


def _sc_gather(frames_flat):
    mesh = plsc.VectorSubcoreMesh(core_axis_name="c", subcore_axis_name="s")

    @functools.partial(
        pl.kernel,
        mesh=mesh,
        out_type=jax.ShapeDtypeStruct((C * TS * FRAME,), jnp.float32),
        scratch_types=[
            pltpu.VMEM((2, QUARTER), jnp.float32),
            pltpu.SemaphoreType.DMA,
            pltpu.SemaphoreType.DMA,
        ],
    )
    def k(src, out, buf, sem_r, sem_w):
        wid = lax.axis_index("s") * 2 + lax.axis_index("c")

        def rd(j):
            p = wid * PER_W + j
            slab = p // 4
            q = p % 4
            c = slab // TS
            i = slab % TS
            src_off = (c * T + (i * 21) // 5) * FRAME + q * QUARTER
            return pltpu.make_async_copy(
                src.at[pl.ds(src_off, QUARTER)], buf.at[j % 2], sem_r
            )

        def wr(j):
            p = wid * PER_W + j
            return pltpu.make_async_copy(
                buf.at[j % 2], out.at[pl.ds(p * QUARTER, QUARTER)], sem_w
            )

        rd(0).start()
        for j in range(PER_W):
            rd(j).wait()
            if j >= 1:
                wr(j - 1).wait()
            wr(j).start()
            if j + 1 < PER_W:
                rd(j + 1).start()
        wr(PER_W - 1).wait()

    return k(frames_flat)


def kernel(frames):
    slow = _sc_gather(frames.reshape(-1)).reshape(C, TS, H, W)
    return (slow, frames)
